# baseline (speedup 1.0000x reference)
import functools

import jax
import jax.numpy as jnp
from jax import lax
from jax.experimental import pallas as pl
from jax.experimental.pallas import tpu as pltpu
from jax.experimental.pallas import tpu_sc as plsc

VOCAB = 100000
WORD_DIM = 300
CHAR_VOCAB = 262
CHAR_DIM = 64
HIDDEN = 128
H2 = HIDDEN // 2
B = 1024
L = 200
W = 16
K = 5
T = W - K + 1

N = B * L

NC = 2
NS = 16
NW = NC * NS
LANES = 16

TOK_PER_W = N // NW
NB = 128
NCHUNK = TOK_PER_W // NB
NG = NB // LANES

ROWS_PCT = K * CHAR_VOCAB
HP = H2 // 2


def _pw_body(wt_ref, wp_ref, o_ref):
    o_ref[...] = jnp.dot(wt_ref[...], wp_ref[...],
                         preferred_element_type=jnp.float32)


def _project_word(word_table, word_proj):
    rows = 1000
    return pl.pallas_call(
        _pw_body,
        grid=(VOCAB // rows,),
        in_specs=[
            pl.BlockSpec((rows, WORD_DIM), lambda i: (i, 0)),
            pl.BlockSpec((WORD_DIM, H2), lambda i: (0, 0)),
        ],
        out_specs=pl.BlockSpec((rows, H2), lambda i: (i, 0)),
        out_shape=jax.ShapeDtypeStruct((VOCAB, H2), jnp.float32),
    )(word_table, word_proj)


def _pct_body(ct_ref, w_ref, b_ref, o_ref):
    k = pl.program_id(0)
    acc = jnp.dot(ct_ref[...], w_ref[0], preferred_element_type=jnp.float32)
    scale = jnp.where(k == 0, 1.0, 0.0)
    o_ref[0] = acc + scale * b_ref[...]


def _char_tables(char_table, char_conv_w, char_conv_b):
    out = pl.pallas_call(
        _pct_body,
        grid=(K,),
        in_specs=[
            pl.BlockSpec((CHAR_VOCAB, CHAR_DIM), lambda k: (0, 0)),
            pl.BlockSpec((1, CHAR_DIM, H2), lambda k: (k, 0, 0)),
            pl.BlockSpec((1, H2), lambda k: (0, 0)),
        ],
        out_specs=pl.BlockSpec((1, CHAR_VOCAB, H2), lambda k: (k, 0, 0)),
        out_shape=jax.ShapeDtypeStruct((K, CHAR_VOCAB, H2), jnp.float32),
    )(char_table, char_conv_w, char_conv_b.reshape(1, H2))
    return out.reshape(ROWS_PCT, H2)


@functools.cache
def _build_sc_main():
    mesh = plsc.VectorSubcoreMesh(core_axis_name="c", subcore_axis_name="s",
                                  num_cores=NC, num_subcores=NS)
    return pl.kernel(
        _sc_body,
        out_type=jax.ShapeDtypeStruct((N * HIDDEN,), jnp.float32),
        mesh=mesh,
        scratch_types=[
            pltpu.VMEM((ROWS_PCT * HP,), jnp.float32),
            pltpu.VMEM((2 * NB,), jnp.int32),
            pltpu.VMEM((2 * NB * W,), jnp.int32),
            pltpu.VMEM((2, NB, H2), jnp.float32),
            pltpu.VMEM((2 * NB * HIDDEN,), jnp.float32),
            pltpu.SemaphoreType.DMA,
            pltpu.SemaphoreType.DMA,
            pltpu.SemaphoreType.DMA,
        ],
        compiler_params=pltpu.CompilerParams(use_tc_tiling_on_sc=False,
                                             needs_layout_passes=False),
    )


def _sc_body(wflat_hbm, cflat_hbm, pw_hbm, pct_hbm, out_hbm,
             pct_v, widx_v, cidx_v, wrows_v, obuf_v, sem_in, sem_g, sem_o):
    wid = lax.axis_index("s") * NC + lax.axis_index("c")
    pltpu.sync_copy(pct_hbm, pct_v)

    def in_copies(ci, b):
        base = wid * TOK_PER_W + ci * NB
        return (
            pltpu.make_async_copy(wflat_hbm.at[pl.ds(base, NB)],
                                  widx_v.at[pl.ds(b * NB, NB)], sem_in),
            pltpu.make_async_copy(cflat_hbm.at[pl.ds(base * W, NB * W)],
                                  cidx_v.at[pl.ds(b * NB * W, NB * W)],
                                  sem_in),
        )

    def gather_copy(b):
        return pltpu.make_async_copy(
            pw_hbm.at[widx_v.at[pl.ds(b * NB, NB)]], wrows_v.at[b], sem_g)

    def out_copy(ci, b):
        base = wid * TOK_PER_W + ci * NB
        return pltpu.make_async_copy(
            obuf_v.at[pl.ds(b * NB * HIDDEN, NB * HIDDEN)],
            out_hbm.at[pl.ds(base * HIDDEN, NB * HIDDEN)], sem_o)

    for c in in_copies(0, 0):
        c.start()
    for c in in_copies(0, 0):
        c.wait()
    gather_copy(0).start()
    for c in in_copies(1, 1):
        c.start()

    def chunk_body(ci, carry):
        b = ci % 2
        gather_copy(b).wait()

        @pl.when(ci >= 2)
        def _():
            out_copy(ci - 2, b).wait()

        def tok_body(i, carry2):
            ob = (b * NB + i) * HIDDEN
            for c4 in range(H2 // LANES):
                obuf_v[pl.ds(ob + c4 * LANES, LANES)] = (
                    wrows_v[b, i, pl.ds(c4 * LANES, LANES)])
            cvec = cidx_v[pl.ds((b * NB + i) * W, W)]
            cj = [cvec[j] * HP for j in range(W)]
            m0 = m1 = None
            for t in range(T):
                s0 = s1 = None
                for k in range(K):
                    adr = cj[t + k] + (k * CHAR_VOCAB * HP)
                    lo = plsc.bitcast(pct_v[pl.ds(adr, LANES)], jnp.bfloat16)
                    hi = plsc.bitcast(pct_v[pl.ds(adr + LANES, LANES)],
                                      jnp.bfloat16)
                    s0 = lo if s0 is None else s0 + lo
                    s1 = hi if s1 is None else s1 + hi
                s0 = jnp.maximum(s0, jnp.bfloat16(0))
                s1 = jnp.maximum(s1, jnp.bfloat16(0))
                m0 = s0 if m0 is None else jnp.maximum(m0, s0)
                m1 = s1 if m1 is None else jnp.maximum(m1, s1)
            a0, b0 = plsc.unpack(m0, format=plsc.PackFormat.INTERLEAVED)
            a1, b1 = plsc.unpack(m1, format=plsc.PackFormat.INTERLEAVED)
            obuf_v[pl.ds(ob + 64, LANES)] = a0
            obuf_v[pl.ds(ob + 80, LANES)] = a1
            obuf_v[pl.ds(ob + 96, LANES)] = b0
            obuf_v[pl.ds(ob + 112, LANES)] = b1
            return carry2

        lax.fori_loop(0, NB, tok_body, 0, unroll=2)
        out_copy(ci, b).start()

        @pl.when(ci + 1 < NCHUNK)
        def _():
            for c in in_copies(ci + 1, 1 - b):
                c.wait()
            gather_copy(1 - b).start()

        @pl.when(ci + 2 < NCHUNK)
        def _():
            for c in in_copies(ci + 2, b):
                c.start()

        return carry

    lax.fori_loop(0, NCHUNK, chunk_body, 0)
    out_copy(NCHUNK - 2, NCHUNK % 2).wait()
    out_copy(NCHUNK - 1, 1 - NCHUNK % 2).wait()


def kernel(w_idxs, c_idxs, word_table, char_table, word_proj,
           char_conv_w, char_conv_b):
    pw = _project_word(word_table, word_proj)
    pct = _char_tables(char_table, char_conv_w, char_conv_b)
    pct = lax.bitcast_convert_type(
        pct.astype(jnp.bfloat16).reshape(ROWS_PCT, 2, HP).transpose(0, 2, 1),
        jnp.float32)
    out = _build_sc_main()(w_idxs.reshape(-1), c_idxs.reshape(-1),
                           pw, pct.reshape(-1))
    return out.reshape(B, L, HIDDEN)


# speedup vs baseline: 1.0577x; 1.0577x over previous
import functools

import jax
import jax.numpy as jnp
from jax import lax
from jax.experimental import pallas as pl
from jax.experimental.pallas import tpu as pltpu
from jax.experimental.pallas import tpu_sc as plsc

VOCAB = 100000
WORD_DIM = 300
CHAR_VOCAB = 262
CHAR_DIM = 64
HIDDEN = 128
H2 = HIDDEN // 2
B = 1024
L = 200
W = 16
K = 5
T = W - K + 1

N = B * L

NC = 2
NS = 16
NW = NC * NS
LANES = 16

TOK_PER_W = N // NW
NB = 128
NCHUNK = TOK_PER_W // NB
NG = NB // LANES

ROWS_PCT = K * CHAR_VOCAB
HP = H2 // 2


def _pw_body(wt_ref, wp_ref, o_ref):
    o_ref[...] = jnp.dot(wt_ref[...], wp_ref[...],
                         preferred_element_type=jnp.float32)


def _project_word(word_table, word_proj):
    rows = 1000
    return pl.pallas_call(
        _pw_body,
        grid=(VOCAB // rows,),
        in_specs=[
            pl.BlockSpec((rows, WORD_DIM), lambda i: (i, 0)),
            pl.BlockSpec((WORD_DIM, H2), lambda i: (0, 0)),
        ],
        out_specs=pl.BlockSpec((rows, H2), lambda i: (i, 0)),
        out_shape=jax.ShapeDtypeStruct((VOCAB, H2), jnp.float32),
    )(word_table, word_proj)


def _pct_body(ct_ref, w_ref, b_ref, o_ref):
    k = pl.program_id(0)
    acc = jnp.dot(ct_ref[...], w_ref[0], preferred_element_type=jnp.float32)
    scale = jnp.where(k == 0, 1.0, 0.0)
    o_ref[0] = acc + scale * b_ref[...]


def _char_tables(char_table, char_conv_w, char_conv_b):
    out = pl.pallas_call(
        _pct_body,
        grid=(K,),
        in_specs=[
            pl.BlockSpec((CHAR_VOCAB, CHAR_DIM), lambda k: (0, 0)),
            pl.BlockSpec((1, CHAR_DIM, H2), lambda k: (k, 0, 0)),
            pl.BlockSpec((1, H2), lambda k: (0, 0)),
        ],
        out_specs=pl.BlockSpec((1, CHAR_VOCAB, H2), lambda k: (k, 0, 0)),
        out_shape=jax.ShapeDtypeStruct((K, CHAR_VOCAB, H2), jnp.float32),
    )(char_table, char_conv_w, char_conv_b.reshape(1, H2))
    return out.reshape(ROWS_PCT, H2)


@functools.cache
def _build_sc_main():
    mesh = plsc.VectorSubcoreMesh(core_axis_name="c", subcore_axis_name="s",
                                  num_cores=NC, num_subcores=NS)
    return pl.kernel(
        _sc_body,
        out_type=jax.ShapeDtypeStruct((N, HIDDEN), jnp.float32),
        mesh=mesh,
        scratch_types=[
            pltpu.VMEM((ROWS_PCT * HP,), jnp.float32),
            pltpu.VMEM((2 * NB,), jnp.int32),
            pltpu.VMEM((2 * NB * W,), jnp.int32),
            pltpu.VMEM((2, NB, H2), jnp.float32),
            pltpu.VMEM((2, NB, H2), jnp.float32),
            pltpu.SemaphoreType.DMA,
            pltpu.SemaphoreType.DMA,
            pltpu.SemaphoreType.DMA,
        ],
        compiler_params=pltpu.CompilerParams(use_tc_tiling_on_sc=False,
                                             needs_layout_passes=False),
    )


def _sc_body(wflat_hbm, cflat_hbm, pw_hbm, pct_hbm, out_hbm,
             pct_v, widx_v, cidx_v, wrows_v, obuf_v, sem_in, sem_g, sem_o):
    wid = lax.axis_index("s") * NC + lax.axis_index("c")
    pltpu.sync_copy(pct_hbm, pct_v)

    def in_copies(ci, b):
        base = wid * TOK_PER_W + ci * NB
        return (
            pltpu.make_async_copy(wflat_hbm.at[pl.ds(base, NB)],
                                  widx_v.at[pl.ds(b * NB, NB)], sem_in),
            pltpu.make_async_copy(cflat_hbm.at[pl.ds(base * W, NB * W)],
                                  cidx_v.at[pl.ds(b * NB * W, NB * W)],
                                  sem_in),
        )

    def gather_copy(b):
        return pltpu.make_async_copy(
            pw_hbm.at[widx_v.at[pl.ds(b * NB, NB)]], wrows_v.at[b], sem_g)

    def out_copies(ci, b):
        base = wid * TOK_PER_W + ci * NB
        return (
            pltpu.make_async_copy(
                wrows_v.at[b],
                out_hbm.at[pl.ds(base, NB), pl.ds(0, H2)], sem_o),
            pltpu.make_async_copy(
                obuf_v.at[b],
                out_hbm.at[pl.ds(base, NB), pl.ds(H2, H2)], sem_o),
        )

    for c in in_copies(0, 0):
        c.start()
    for c in in_copies(0, 0):
        c.wait()
    gather_copy(0).start()
    for c in in_copies(1, 1):
        c.start()

    def chunk_body(ci, carry):
        b = ci % 2
        gather_copy(b).wait()

        def tok_body(i, carry2):
            cvec = cidx_v[pl.ds((b * NB + i) * W, W)]
            cj = [cvec[j] * HP for j in range(W)]
            m0 = m1 = None
            for t in range(T):
                s0 = s1 = None
                for k in range(K):
                    adr = cj[t + k] + (k * CHAR_VOCAB * HP)
                    lo = plsc.bitcast(pct_v[pl.ds(adr, LANES)], jnp.bfloat16)
                    hi = plsc.bitcast(pct_v[pl.ds(adr + LANES, LANES)],
                                      jnp.bfloat16)
                    s0 = lo if s0 is None else s0 + lo
                    s1 = hi if s1 is None else s1 + hi
                s0 = jnp.maximum(s0, jnp.bfloat16(0))
                s1 = jnp.maximum(s1, jnp.bfloat16(0))
                m0 = s0 if m0 is None else jnp.maximum(m0, s0)
                m1 = s1 if m1 is None else jnp.maximum(m1, s1)
            a0, b0 = plsc.unpack(m0, format=plsc.PackFormat.INTERLEAVED)
            a1, b1 = plsc.unpack(m1, format=plsc.PackFormat.INTERLEAVED)
            obuf_v[b, i, pl.ds(0, LANES)] = a0
            obuf_v[b, i, pl.ds(16, LANES)] = a1
            obuf_v[b, i, pl.ds(32, LANES)] = b0
            obuf_v[b, i, pl.ds(48, LANES)] = b1
            return carry2

        lax.fori_loop(0, NB, tok_body, 0)
        for c in out_copies(ci, b):
            c.start()

        @pl.when(ci + 1 < NCHUNK)
        def _():
            for c in in_copies(ci + 1, 1 - b):
                c.wait()

            @pl.when(ci >= 1)
            def _():
                for c in out_copies(ci - 1, 1 - b):
                    c.wait()

            gather_copy(1 - b).start()

        @pl.when(ci + 2 < NCHUNK)
        def _():
            for c in in_copies(ci + 2, b):
                c.start()

        return carry

    lax.fori_loop(0, NCHUNK, chunk_body, 0)
    for c in out_copies(NCHUNK - 2, NCHUNK % 2):
        c.wait()
    for c in out_copies(NCHUNK - 1, 1 - NCHUNK % 2):
        c.wait()


def kernel(w_idxs, c_idxs, word_table, char_table, word_proj,
           char_conv_w, char_conv_b):
    pw = _project_word(word_table, word_proj)
    pct = _char_tables(char_table, char_conv_w, char_conv_b)
    pct = lax.bitcast_convert_type(
        pct.astype(jnp.bfloat16).reshape(ROWS_PCT, 2, HP).transpose(0, 2, 1),
        jnp.float32)
    out = _build_sc_main()(w_idxs.reshape(-1), c_idxs.reshape(-1),
                           pw, pct.reshape(-1))
    return out.reshape(B, L, HIDDEN)


# speedup vs baseline: 1.3678x; 1.2933x over previous
import functools

import jax
import jax.numpy as jnp
from jax import lax
from jax.experimental import pallas as pl
from jax.experimental.pallas import tpu as pltpu
from jax.experimental.pallas import tpu_sc as plsc

VOCAB = 100000
WORD_DIM = 300
CHAR_VOCAB = 262
CHAR_DIM = 64
HIDDEN = 128
H2 = HIDDEN // 2
B = 1024
L = 200
W = 16
K = 5
T = W - K + 1

N = B * L

NC = 2
NS = 16
NW = NC * NS
LANES = 16

TOK_PER_W = N // NW
NB = 128
NCHUNK = TOK_PER_W // NB
NG = NB // LANES

ROWS_PCT = K * CHAR_VOCAB
HP = H2 // 2


def _pw_body(wt_ref, wp_ref, o_ref):
    o_ref[...] = jnp.dot(wt_ref[...], wp_ref[...],
                         preferred_element_type=jnp.float32)


def _project_word(word_table, word_proj):
    rows = 1000
    return pl.pallas_call(
        _pw_body,
        grid=(VOCAB // rows,),
        in_specs=[
            pl.BlockSpec((rows, WORD_DIM), lambda i: (i, 0)),
            pl.BlockSpec((WORD_DIM, H2), lambda i: (0, 0)),
        ],
        out_specs=pl.BlockSpec((rows, H2), lambda i: (i, 0)),
        out_shape=jax.ShapeDtypeStruct((VOCAB, H2), jnp.float32),
    )(word_table, word_proj)


def _pct_body(ct_ref, w_ref, b_ref, o_ref):
    k = pl.program_id(0)
    acc = jnp.dot(ct_ref[...], w_ref[0], preferred_element_type=jnp.float32)
    scale = jnp.where(k == 0, 1.0, 0.0)
    o_ref[0] = acc + scale * b_ref[...]


def _char_tables(char_table, char_conv_w, char_conv_b):
    out = pl.pallas_call(
        _pct_body,
        grid=(K,),
        in_specs=[
            pl.BlockSpec((CHAR_VOCAB, CHAR_DIM), lambda k: (0, 0)),
            pl.BlockSpec((1, CHAR_DIM, H2), lambda k: (k, 0, 0)),
            pl.BlockSpec((1, H2), lambda k: (0, 0)),
        ],
        out_specs=pl.BlockSpec((1, CHAR_VOCAB, H2), lambda k: (k, 0, 0)),
        out_shape=jax.ShapeDtypeStruct((K, CHAR_VOCAB, H2), jnp.float32),
    )(char_table, char_conv_w, char_conv_b.reshape(1, H2))
    return out.reshape(ROWS_PCT, H2)


_SC_PARAMS = dict(
    compiler_params=pltpu.CompilerParams(use_tc_tiling_on_sc=False,
                                         needs_layout_passes=False),
)


def _sc_mesh():
    return plsc.VectorSubcoreMesh(core_axis_name="c", subcore_axis_name="s",
                                  num_cores=NC, num_subcores=NS)


@functools.cache
def _build_sc_char():
    return pl.kernel(
        _sc_char_body,
        out_type=jax.ShapeDtypeStruct((N, H2), jnp.float32),
        mesh=_sc_mesh(),
        scratch_types=[
            pltpu.VMEM((ROWS_PCT * HP,), jnp.float32),
            pltpu.VMEM((2 * NB * W,), jnp.int32),
            pltpu.VMEM((2, NB, H2), jnp.float32),
            pltpu.SemaphoreType.DMA,
            pltpu.SemaphoreType.DMA,
        ],
        **_SC_PARAMS,
    )


def _sc_char_body(cflat_hbm, pct_hbm, cp_hbm, pct_v, cidx_v, obuf_v,
                  sem_in, sem_o):
    wid = lax.axis_index("s") * NC + lax.axis_index("c")
    pltpu.sync_copy(pct_hbm, pct_v)

    def in_copy(ci, b):
        base = wid * TOK_PER_W + ci * NB
        return pltpu.make_async_copy(
            cflat_hbm.at[pl.ds(base * W, NB * W)],
            cidx_v.at[pl.ds(b * NB * W, NB * W)], sem_in)

    def out_copy(ci, b):
        base = wid * TOK_PER_W + ci * NB
        return pltpu.make_async_copy(
            obuf_v.at[b], cp_hbm.at[pl.ds(base, NB)], sem_o)

    in_copy(0, 0).start()
    in_copy(0, 0).wait()
    in_copy(1, 1).start()

    def chunk_body(ci, carry):
        b = ci % 2

        @pl.when(ci >= 2)
        def _():
            out_copy(ci - 2, b).wait()

        def tok_body(i, carry2):
            cvec = cidx_v[pl.ds((b * NB + i) * W, W)]
            cj = [cvec[j] * HP for j in range(W)]
            m0 = m1 = None
            for t in range(T):
                s0 = s1 = None
                for k in range(K):
                    adr = cj[t + k] + (k * CHAR_VOCAB * HP)
                    lo = plsc.bitcast(pct_v[pl.ds(adr, LANES)], jnp.bfloat16)
                    hi = plsc.bitcast(pct_v[pl.ds(adr + LANES, LANES)],
                                      jnp.bfloat16)
                    s0 = lo if s0 is None else s0 + lo
                    s1 = hi if s1 is None else s1 + hi
                s0 = jnp.maximum(s0, jnp.bfloat16(0))
                s1 = jnp.maximum(s1, jnp.bfloat16(0))
                m0 = s0 if m0 is None else jnp.maximum(m0, s0)
                m1 = s1 if m1 is None else jnp.maximum(m1, s1)
            a0, b0 = plsc.unpack(m0, format=plsc.PackFormat.INTERLEAVED)
            a1, b1 = plsc.unpack(m1, format=plsc.PackFormat.INTERLEAVED)
            obuf_v[b, i, pl.ds(0, LANES)] = a0
            obuf_v[b, i, pl.ds(16, LANES)] = a1
            obuf_v[b, i, pl.ds(32, LANES)] = b0
            obuf_v[b, i, pl.ds(48, LANES)] = b1
            return carry2

        lax.fori_loop(0, NB, tok_body, 0)
        out_copy(ci, b).start()

        @pl.when(ci + 1 < NCHUNK)
        def _():
            in_copy(ci + 1, 1 - b).wait()

        @pl.when(ci + 2 < NCHUNK)
        def _():
            in_copy(ci + 2, b).start()

        return carry

    lax.fori_loop(0, NCHUNK, chunk_body, 0)
    out_copy(NCHUNK - 2, NCHUNK % 2).wait()
    out_copy(NCHUNK - 1, 1 - NCHUNK % 2).wait()


@functools.cache
def _build_sc_word():
    return pl.kernel(
        _sc_word_body,
        out_type=jax.ShapeDtypeStruct((N, HIDDEN), jnp.float32),
        mesh=_sc_mesh(),
        scratch_types=[
            pltpu.VMEM((2 * NB,), jnp.int32),
            pltpu.VMEM((2, NB, H2), jnp.float32),
            pltpu.VMEM((4, NB, H2), jnp.float32),
            pltpu.SemaphoreType.DMA,
            pltpu.SemaphoreType.DMA,
            pltpu.SemaphoreType.DMA,
        ],
        **_SC_PARAMS,
    )


def _sc_word_body(wflat_hbm, pw_hbm, cp_hbm, out_hbm,
                  widx_v, wrows_v, cbuf_v, sem_in, sem_g, sem_o):
    wid = lax.axis_index("s") * NC + lax.axis_index("c")

    def in_copies(ci, b, b4):
        base = wid * TOK_PER_W + ci * NB
        return (
            pltpu.make_async_copy(wflat_hbm.at[pl.ds(base, NB)],
                                  widx_v.at[pl.ds(b * NB, NB)], sem_in),
            pltpu.make_async_copy(cp_hbm.at[pl.ds(base, NB)],
                                  cbuf_v.at[b4], sem_in),
        )

    def gather_copy(b):
        return pltpu.make_async_copy(
            pw_hbm.at[widx_v.at[pl.ds(b * NB, NB)]], wrows_v.at[b], sem_g)

    def out_copies(ci, b, b4):
        base = wid * TOK_PER_W + ci * NB
        return (
            pltpu.make_async_copy(
                wrows_v.at[b],
                out_hbm.at[pl.ds(base, NB), pl.ds(0, H2)], sem_o),
            pltpu.make_async_copy(
                cbuf_v.at[b4],
                out_hbm.at[pl.ds(base, NB), pl.ds(H2, H2)], sem_o),
        )

    for c in in_copies(0, 0, 0):
        c.start()
    for c in in_copies(0, 0, 0):
        c.wait()
    gather_copy(0).start()
    for c in in_copies(1, 1, 1):
        c.start()

    def chunk_body(ci, carry):
        b = ci % 2
        b4 = ci % 4
        gather_copy(b).wait()
        for c in out_copies(ci, b, b4):
            c.start()

        @pl.when(ci + 1 < NCHUNK)
        def _():
            for c in in_copies(ci + 1, 1 - b, (ci + 1) % 4):
                c.wait()

            @pl.when(ci >= 1)
            def _():
                for c in out_copies(ci - 1, 1 - b, (ci - 1) % 4):
                    c.wait()

            gather_copy(1 - b).start()

        @pl.when(ci + 2 < NCHUNK)
        def _():
            for c in in_copies(ci + 2, b, (ci + 2) % 4):
                c.start()

        return carry

    lax.fori_loop(0, NCHUNK, chunk_body, 0)
    for c in out_copies(NCHUNK - 2, NCHUNK % 2, (NCHUNK - 2) % 4):
        c.wait()
    for c in out_copies(NCHUNK - 1, 1 - NCHUNK % 2, (NCHUNK - 1) % 4):
        c.wait()


def kernel(w_idxs, c_idxs, word_table, char_table, word_proj,
           char_conv_w, char_conv_b):
    pw = _project_word(word_table, word_proj)
    pct = _char_tables(char_table, char_conv_w, char_conv_b)
    pct = lax.bitcast_convert_type(
        pct.astype(jnp.bfloat16).reshape(ROWS_PCT, 2, HP).transpose(0, 2, 1),
        jnp.float32)
    cp = _build_sc_char()(c_idxs.reshape(-1), pct.reshape(-1))
    out = _build_sc_word()(w_idxs.reshape(-1), pw, cp)
    return out.reshape(B, L, HIDDEN)


# speedup vs baseline: 1.4186x; 1.0372x over previous
import functools

import jax
import jax.numpy as jnp
from jax import lax
from jax.experimental import pallas as pl
from jax.experimental.pallas import tpu as pltpu
from jax.experimental.pallas import tpu_sc as plsc

VOCAB = 100000
WORD_DIM = 300
CHAR_VOCAB = 262
CHAR_DIM = 64
HIDDEN = 128
H2 = HIDDEN // 2
B = 1024
L = 200
W = 16
K = 5
T = W - K + 1

N = B * L

NC = 2
NS = 16
NW = NC * NS
LANES = 16

TOK_PER_W = N // NW
NB = 128
NCHUNK = TOK_PER_W // NB
NG = NB // LANES

ROWS_PCT = K * CHAR_VOCAB
HP = H2 // 2


def _pw_body(wt_ref, wp_ref, o_ref):
    o_ref[...] = jnp.dot(wt_ref[...], wp_ref[...],
                         preferred_element_type=jnp.float32)


def _project_word(word_table, word_proj):
    rows = 1000
    return pl.pallas_call(
        _pw_body,
        grid=(VOCAB // rows,),
        in_specs=[
            pl.BlockSpec((rows, WORD_DIM), lambda i: (i, 0)),
            pl.BlockSpec((WORD_DIM, H2), lambda i: (0, 0)),
        ],
        out_specs=pl.BlockSpec((rows, H2), lambda i: (i, 0)),
        out_shape=jax.ShapeDtypeStruct((VOCAB, H2), jnp.float32),
    )(word_table, word_proj)


def _pct_body(ct_ref, w_ref, b_ref, o_ref):
    k = pl.program_id(0)
    acc = jnp.dot(ct_ref[...], w_ref[0], preferred_element_type=jnp.float32)
    scale = jnp.where(k == 0, 1.0, 0.0)
    o_ref[0] = acc + scale * b_ref[...]


def _char_tables(char_table, char_conv_w, char_conv_b):
    out = pl.pallas_call(
        _pct_body,
        grid=(K,),
        in_specs=[
            pl.BlockSpec((CHAR_VOCAB, CHAR_DIM), lambda k: (0, 0)),
            pl.BlockSpec((1, CHAR_DIM, H2), lambda k: (k, 0, 0)),
            pl.BlockSpec((1, H2), lambda k: (0, 0)),
        ],
        out_specs=pl.BlockSpec((1, CHAR_VOCAB, H2), lambda k: (k, 0, 0)),
        out_shape=jax.ShapeDtypeStruct((K, CHAR_VOCAB, H2), jnp.float32),
    )(char_table, char_conv_w, char_conv_b.reshape(1, H2))
    return out.reshape(ROWS_PCT, H2)


_SC_PARAMS = dict(
    compiler_params=pltpu.CompilerParams(use_tc_tiling_on_sc=False,
                                         needs_layout_passes=False),
)


def _sc_mesh():
    return plsc.VectorSubcoreMesh(core_axis_name="c", subcore_axis_name="s",
                                  num_cores=NC, num_subcores=NS)


@functools.cache
def _build_sc_char():
    return pl.kernel(
        _sc_char_body,
        out_type=jax.ShapeDtypeStruct((N, H2), jnp.float32),
        mesh=_sc_mesh(),
        scratch_types=[
            pltpu.VMEM((ROWS_PCT * HP,), jnp.float32),
            pltpu.VMEM((2 * NB * W,), jnp.int32),
            pltpu.VMEM((2, NB, H2), jnp.float32),
            pltpu.SemaphoreType.DMA,
            pltpu.SemaphoreType.DMA,
        ],
        **_SC_PARAMS,
    )


def _sc_char_body(cflat_hbm, pct_hbm, cp_hbm, pct_v, cidx_v, obuf_v,
                  sem_in, sem_o):
    wid = lax.axis_index("s") * NC + lax.axis_index("c")
    pltpu.sync_copy(pct_hbm, pct_v)

    def in_copy(ci, b):
        base = wid * TOK_PER_W + ci * NB
        return pltpu.make_async_copy(
            cflat_hbm.at[pl.ds(base * W, NB * W)],
            cidx_v.at[pl.ds(b * NB * W, NB * W)], sem_in)

    def out_copy(ci, b):
        base = wid * TOK_PER_W + ci * NB
        return pltpu.make_async_copy(
            obuf_v.at[b], cp_hbm.at[pl.ds(base, NB)], sem_o)

    in_copy(0, 0).start()
    in_copy(0, 0).wait()
    in_copy(1, 1).start()

    def chunk_body(ci, carry):
        b = ci % 2

        @pl.when(ci >= 2)
        def _():
            out_copy(ci - 2, b).wait()

        def tok_body(i, carry2):
            cvec = cidx_v[pl.ds((b * NB + i) * W, W)]
            cj = [cvec[j] * HP for j in range(W)]
            m0 = m1 = None
            for t in range(T):
                s0 = s1 = None
                for k in range(K):
                    adr = cj[t + k] + (k * CHAR_VOCAB * HP)
                    lo = plsc.bitcast(pct_v[pl.ds(adr, LANES)], jnp.bfloat16)
                    hi = plsc.bitcast(pct_v[pl.ds(adr + LANES, LANES)],
                                      jnp.bfloat16)
                    s0 = lo if s0 is None else s0 + lo
                    s1 = hi if s1 is None else s1 + hi
                s0 = jnp.maximum(s0, jnp.bfloat16(0))
                s1 = jnp.maximum(s1, jnp.bfloat16(0))
                m0 = s0 if m0 is None else jnp.maximum(m0, s0)
                m1 = s1 if m1 is None else jnp.maximum(m1, s1)
            a0, b0 = plsc.unpack(m0, format=plsc.PackFormat.INTERLEAVED)
            a1, b1 = plsc.unpack(m1, format=plsc.PackFormat.INTERLEAVED)
            obuf_v[b, i, pl.ds(0, LANES)] = a0
            obuf_v[b, i, pl.ds(16, LANES)] = a1
            obuf_v[b, i, pl.ds(32, LANES)] = b0
            obuf_v[b, i, pl.ds(48, LANES)] = b1
            return carry2

        lax.fori_loop(0, NB, tok_body, 0, unroll=2)
        out_copy(ci, b).start()

        @pl.when(ci + 1 < NCHUNK)
        def _():
            in_copy(ci + 1, 1 - b).wait()

        @pl.when(ci + 2 < NCHUNK)
        def _():
            in_copy(ci + 2, b).start()

        return carry

    lax.fori_loop(0, NCHUNK, chunk_body, 0)
    out_copy(NCHUNK - 2, NCHUNK % 2).wait()
    out_copy(NCHUNK - 1, 1 - NCHUNK % 2).wait()


@functools.cache
def _build_sc_word():
    return pl.kernel(
        _sc_word_body,
        out_type=jax.ShapeDtypeStruct((N, HIDDEN), jnp.float32),
        mesh=_sc_mesh(),
        scratch_types=[
            pltpu.VMEM((2 * NB,), jnp.int32),
            pltpu.VMEM((2, NB, H2), jnp.float32),
            pltpu.VMEM((4, NB, H2), jnp.float32),
            pltpu.SemaphoreType.DMA,
            pltpu.SemaphoreType.DMA,
            pltpu.SemaphoreType.DMA,
        ],
        **_SC_PARAMS,
    )


def _sc_word_body(wflat_hbm, pw_hbm, cp_hbm, out_hbm,
                  widx_v, wrows_v, cbuf_v, sem_in, sem_g, sem_o):
    wid = lax.axis_index("s") * NC + lax.axis_index("c")

    def in_copies(ci, b, b4):
        base = wid * TOK_PER_W + ci * NB
        return (
            pltpu.make_async_copy(wflat_hbm.at[pl.ds(base, NB)],
                                  widx_v.at[pl.ds(b * NB, NB)], sem_in),
            pltpu.make_async_copy(cp_hbm.at[pl.ds(base, NB)],
                                  cbuf_v.at[b4], sem_in),
        )

    def gather_copy(b):
        return pltpu.make_async_copy(
            pw_hbm.at[widx_v.at[pl.ds(b * NB, NB)]], wrows_v.at[b], sem_g)

    def out_copies(ci, b, b4):
        base = wid * TOK_PER_W + ci * NB
        return (
            pltpu.make_async_copy(
                wrows_v.at[b],
                out_hbm.at[pl.ds(base, NB), pl.ds(0, H2)], sem_o),
            pltpu.make_async_copy(
                cbuf_v.at[b4],
                out_hbm.at[pl.ds(base, NB), pl.ds(H2, H2)], sem_o),
        )

    for c in in_copies(0, 0, 0):
        c.start()
    for c in in_copies(0, 0, 0):
        c.wait()
    gather_copy(0).start()
    for c in in_copies(1, 1, 1):
        c.start()

    def chunk_body(ci, carry):
        b = ci % 2
        b4 = ci % 4
        gather_copy(b).wait()
        for c in out_copies(ci, b, b4):
            c.start()

        @pl.when(ci + 1 < NCHUNK)
        def _():
            for c in in_copies(ci + 1, 1 - b, (ci + 1) % 4):
                c.wait()

            @pl.when(ci >= 1)
            def _():
                for c in out_copies(ci - 1, 1 - b, (ci - 1) % 4):
                    c.wait()

            gather_copy(1 - b).start()

        @pl.when(ci + 2 < NCHUNK)
        def _():
            for c in in_copies(ci + 2, b, (ci + 2) % 4):
                c.start()

        return carry

    lax.fori_loop(0, NCHUNK, chunk_body, 0)
    for c in out_copies(NCHUNK - 2, NCHUNK % 2, (NCHUNK - 2) % 4):
        c.wait()
    for c in out_copies(NCHUNK - 1, 1 - NCHUNK % 2, (NCHUNK - 1) % 4):
        c.wait()


def kernel(w_idxs, c_idxs, word_table, char_table, word_proj,
           char_conv_w, char_conv_b):
    pw = _project_word(word_table, word_proj)
    pct = _char_tables(char_table, char_conv_w, char_conv_b)
    pct = lax.bitcast_convert_type(
        pct.astype(jnp.bfloat16).reshape(ROWS_PCT, 2, HP).transpose(0, 2, 1),
        jnp.float32)
    cp = _build_sc_char()(c_idxs.reshape(-1), pct.reshape(-1))
    out = _build_sc_word()(w_idxs.reshape(-1), pw, cp)
    return out.reshape(B, L, HIDDEN)
